# baseline (speedup 1.0000x reference)
import jax
import jax.numpy as jnp
from jax import lax
from jax.experimental import pallas as pl
from jax.experimental.pallas import tpu as pltpu
from jax.experimental.pallas import tpu_sc as plsc

N_NODES = 10000
N_EDGES = 320000
D_FEAT = 128
N_SUBCORES = 16
FPT = D_FEAT // N_SUBCORES
NP = 10112
SEG = FPT * NP
CHUNK = 128
GRP = 2
CH_PER_SC = 1280
E_PAD = 2 * CH_PER_SC * CHUNK
PASSES = ((0, 5120), (5120, 4992))
ACC_W = 5120


def _zero_1d(ref, n):
    def body(i, carry):
        ref[pl.ds(i * 16, 16)] = jnp.zeros((16,), jnp.float32)
        return carry
    lax.fori_loop(0, n // 16, body, 0)


def _agg_body(xflat, srcflat, dstflat, agg_out, cnt_out,
              x_strip, acc, cnt, idx_s, idx_d, with_counts):
    c = lax.axis_index("c")
    s = lax.axis_index("s")
    pltpu.sync_copy(xflat.at[pl.ds(s * SEG, SEG)], x_strip)
    e0c = c * (CH_PER_SC * CHUNK)
    ones = jnp.ones((16,), jnp.float32)

    for lo, pn in PASSES:
        hi = lo + pn
        _zero_1d(acc, FPT * ACC_W)
        if with_counts:
            _zero_1d(cnt, ACC_W)

        def body(g, carry):
            e0 = e0c + g * (GRP * CHUNK)
            pltpu.sync_copy(srcflat.at[pl.ds(e0, GRP * CHUNK)], idx_s)
            pltpu.sync_copy(dstflat.at[pl.ds(e0, GRP * CHUNK)], idx_d)
            for j in range(GRP * CHUNK // 16):
                src16 = idx_s[pl.ds(j * 16, 16)]
                dst16 = idx_d[pl.ds(j * 16, 16)]
                col = dst16 - lo
                m = (dst16 >= lo) & (dst16 < hi)
                colm = jnp.where(m, col, 0)
                for f in range(FPT):
                    vals = plsc.load_gather(x_strip, [src16 + (f * NP)])
                    plsc.addupdate_scatter(acc, [colm + (f * ACC_W)], vals,
                                           mask=m)
                if with_counts:
                    mc = m & ((g % N_SUBCORES) == s)
                    plsc.addupdate_scatter(cnt, [colm], ones, mask=mc)
            return carry

        lax.fori_loop(0, CH_PER_SC // GRP, body, 0)
        base = (c * N_SUBCORES + s) * SEG
        for f in range(FPT):
            pltpu.sync_copy(acc.at[pl.ds(f * ACC_W, pn)],
                            agg_out.at[pl.ds(base + f * NP + lo, pn)])
        if with_counts:
            pltpu.sync_copy(cnt.at[pl.ds(0, pn)],
                            cnt_out.at[pl.ds((c * N_SUBCORES + s) * NP + lo,
                                             pn)])


def _agg1_body(xflat, srcflat, dstflat, agg_out, cnt_out,
               x_strip, acc, cnt, idx_s, idx_d):
    _agg_body(xflat, srcflat, dstflat, agg_out, cnt_out,
              x_strip, acc, cnt, idx_s, idx_d, True)


def _agg2_body(xflat, srcflat, dstflat, agg_out,
               x_strip, acc, idx_s, idx_d):
    _agg_body(xflat, srcflat, dstflat, agg_out, None,
              x_strip, acc, None, idx_s, idx_d, False)


def _sc_mesh():
    return plsc.VectorSubcoreMesh(core_axis_name="c", subcore_axis_name="s")


_SC_PARAMS = pltpu.CompilerParams(needs_layout_passes=False)


_sc_agg1 = pl.kernel(
    _agg1_body,
    out_type=(
        jax.ShapeDtypeStruct((2 * N_SUBCORES * SEG,), jnp.float32),
        jax.ShapeDtypeStruct((2 * N_SUBCORES * NP,), jnp.float32),
    ),
    mesh=_sc_mesh(),
    compiler_params=_SC_PARAMS,
    scratch_types=[
        pltpu.VMEM((SEG,), jnp.float32),
        pltpu.VMEM((FPT * ACC_W,), jnp.float32),
        pltpu.VMEM((ACC_W,), jnp.float32),
        pltpu.VMEM((GRP * CHUNK,), jnp.int32),
        pltpu.VMEM((GRP * CHUNK,), jnp.int32),
    ],
)

_sc_agg2 = pl.kernel(
    _agg2_body,
    out_type=jax.ShapeDtypeStruct((2 * N_SUBCORES * SEG,), jnp.float32),
    mesh=_sc_mesh(),
    compiler_params=_SC_PARAMS,
    scratch_types=[
        pltpu.VMEM((SEG,), jnp.float32),
        pltpu.VMEM((FPT * ACC_W,), jnp.float32),
        pltpu.VMEM((GRP * CHUNK,), jnp.int32),
        pltpu.VMEM((GRP * CHUNK,), jnp.int32),
    ],
)


def _ddg(a, b):
    return lax.dot_general(a, b, (((0,), (0,)), ((), ())),
                           preferred_element_type=jnp.float32)


def _tc1_body(xT_r, agg0_r, agg1_r, cnt_r, W1l_r, b1l_r, W1r_r, out_r):
    cnt = jnp.sum(cnt_r[...], axis=0, keepdims=True)
    inv = 1.0 / jnp.maximum(cnt, 1.0)
    aggT = agg0_r[...] + agg1_r[...]
    meanT = aggT * inv
    hT = _ddg(W1l_r[...], meanT) + b1l_r[...] + _ddg(W1r_r[...], xT_r[...])
    out_r[...] = jnp.maximum(hT, 0.0)


def _tc1(xT, agg0, agg1, cnt32, W1l, b1l, W1r):
    return pl.pallas_call(
        _tc1_body,
        out_shape=jax.ShapeDtypeStruct((D_FEAT, NP), jnp.float32),
    )(xT, agg0, agg1, cnt32, W1l, b1l.reshape(-1, 1), W1r)


def _tc2_body(h1T_r, agg0_r, agg1_r, cnt_r,
              W2l_r, b2l_r, W2r_r, Wlin_r, blin_r, out_r):
    cnt = jnp.sum(cnt_r[...], axis=0, keepdims=True)
    inv = 1.0 / jnp.maximum(cnt, 1.0)
    meanT = (agg0_r[...] + agg1_r[...]) * inv
    h2T = _ddg(W2l_r[...], meanT) + b2l_r[...] + _ddg(W2r_r[...], h1T_r[...])
    h2T = jnp.maximum(h2T, 0.0)
    out_r[...] = _ddg(h2T, Wlin_r[...]) + blin_r[...]


def _tc2(h1T, agg0, agg1, cnt32, W2l, b2l, W2r, Wlin, blin):
    n_cls = Wlin.shape[1]
    return pl.pallas_call(
        _tc2_body,
        out_shape=jax.ShapeDtypeStruct((NP, n_cls), jnp.float32),
    )(h1T, agg0, agg1, cnt32, W2l, b2l.reshape(-1, 1), W2r,
      Wlin, blin.reshape(1, -1))


def kernel(x, edge_index, W1l, b1l, W1r, W2l, b2l, W2r, Wlin, blin):
    src = edge_index[0].astype(jnp.int32)
    dst = edge_index[1].astype(jnp.int32)
    pad = E_PAD - N_EDGES
    srcflat = jnp.concatenate([src, jnp.zeros((pad,), jnp.int32)])
    dstflat = jnp.concatenate([dst, jnp.full((pad,), N_NODES, jnp.int32)])

    xT = jnp.concatenate(
        [x.T, jnp.zeros((D_FEAT, NP - N_NODES), jnp.float32)], axis=1)
    xflat = xT.reshape(-1)

    aggf, cntf = _sc_agg1(xflat, srcflat, dstflat)
    agg = aggf.reshape(2, D_FEAT, NP)
    cnt32 = cntf.reshape(2 * N_SUBCORES, NP)

    h1T = _tc1(xT, agg[0], agg[1], cnt32, W1l, b1l, W1r)

    agg2f = _sc_agg2(h1T.reshape(-1), srcflat, dstflat)
    agg2 = agg2f.reshape(2, D_FEAT, NP)

    out = _tc2(h1T, agg2[0], agg2[1], cnt32, W2l, b2l, W2r, Wlin, blin)
    return out[:N_NODES]


# speedup vs baseline: 1.7735x; 1.7735x over previous
import jax
import jax.numpy as jnp
from jax import lax
from jax.experimental import pallas as pl
from jax.experimental.pallas import tpu as pltpu
from jax.experimental.pallas import tpu_sc as plsc

N_NODES = 10000
N_EDGES = 320000
D_FEAT = 128
N_SUBCORES = 16
N_TEC = 2 * N_SUBCORES
FPT = D_FEAT // N_TEC
NP = 10112
SEG = FPT * NP
CHUNK = 128
GRP = 4
CH_TOTAL = 2560
E_PAD = CH_TOTAL * CHUNK


def _zero_1d(ref, n):
    def body(i, carry):
        ref[pl.ds(i * 16, 16)] = jnp.zeros((16,), jnp.float32)
        return carry
    lax.fori_loop(0, n // 16, body, 0)


def _agg_body(xflat, srcflat, dstflat, agg_out, cnt_out,
              x_strip, acc, cnt, idx_s, idx_d, with_counts):
    c = lax.axis_index("c")
    s = lax.axis_index("s")
    tec = c * N_SUBCORES + s
    pltpu.sync_copy(xflat.at[pl.ds(tec * SEG, SEG)], x_strip)
    ones = jnp.ones((16,), jnp.float32)

    _zero_1d(acc, SEG)
    if with_counts:
        @pl.when(c == 0)
        def _():
            _zero_1d(cnt, NP)

    def body(g, carry):
        e0 = g * (GRP * CHUNK)
        pltpu.sync_copy(srcflat.at[pl.ds(e0, GRP * CHUNK)], idx_s)
        pltpu.sync_copy(dstflat.at[pl.ds(e0, GRP * CHUNK)], idx_d)
        for j in range(GRP * CHUNK // 16):
            src16 = idx_s[pl.ds(j * 16, 16)]
            dst16 = idx_d[pl.ds(j * 16, 16)]
            for f in range(FPT):
                vals = plsc.load_gather(x_strip, [src16 + (f * NP)])
                plsc.addupdate_scatter(acc, [dst16 + (f * NP)], vals)
            if with_counts:
                mc = ((g % N_SUBCORES) == s) & (c == 0)
                plsc.addupdate_scatter(cnt, [dst16], ones,
                                       mask=jnp.broadcast_to(mc, (16,)))
        return carry

    lax.fori_loop(0, CH_TOTAL // GRP, body, 0)
    pltpu.sync_copy(acc, agg_out.at[pl.ds(tec * SEG, SEG)])
    if with_counts:
        @pl.when(c == 0)
        def _():
            pltpu.sync_copy(cnt, cnt_out.at[pl.ds(s * NP, NP)])


def _agg1_body(xflat, srcflat, dstflat, agg_out, cnt_out,
               x_strip, acc, cnt, idx_s, idx_d):
    _agg_body(xflat, srcflat, dstflat, agg_out, cnt_out,
              x_strip, acc, cnt, idx_s, idx_d, True)


def _agg2_body(xflat, srcflat, dstflat, agg_out,
               x_strip, acc, idx_s, idx_d):
    _agg_body(xflat, srcflat, dstflat, agg_out, None,
              x_strip, acc, None, idx_s, idx_d, False)


def _sc_mesh():
    return plsc.VectorSubcoreMesh(core_axis_name="c", subcore_axis_name="s")


_SC_PARAMS = pltpu.CompilerParams(needs_layout_passes=False)

_sc_agg1 = pl.kernel(
    _agg1_body,
    out_type=(
        jax.ShapeDtypeStruct((D_FEAT * NP,), jnp.float32),
        jax.ShapeDtypeStruct((N_SUBCORES * NP,), jnp.float32),
    ),
    mesh=_sc_mesh(),
    compiler_params=_SC_PARAMS,
    scratch_types=[
        pltpu.VMEM((SEG,), jnp.float32),
        pltpu.VMEM((SEG,), jnp.float32),
        pltpu.VMEM((NP,), jnp.float32),
        pltpu.VMEM((GRP * CHUNK,), jnp.int32),
        pltpu.VMEM((GRP * CHUNK,), jnp.int32),
    ],
)

_sc_agg2 = pl.kernel(
    _agg2_body,
    out_type=jax.ShapeDtypeStruct((D_FEAT * NP,), jnp.float32),
    mesh=_sc_mesh(),
    compiler_params=_SC_PARAMS,
    scratch_types=[
        pltpu.VMEM((SEG,), jnp.float32),
        pltpu.VMEM((SEG,), jnp.float32),
        pltpu.VMEM((GRP * CHUNK,), jnp.int32),
        pltpu.VMEM((GRP * CHUNK,), jnp.int32),
    ],
)


def _ddg(a, b):
    return lax.dot_general(a, b, (((0,), (0,)), ((), ())),
                           preferred_element_type=jnp.float32)


def _tc1_body(xT_r, agg_r, cnt_r, W1l_r, b1l_r, W1r_r, out_r):
    cnt = jnp.sum(cnt_r[...], axis=0, keepdims=True)
    inv = 1.0 / jnp.maximum(cnt, 1.0)
    meanT = agg_r[...] * inv
    hT = _ddg(W1l_r[...], meanT) + b1l_r[...] + _ddg(W1r_r[...], xT_r[...])
    out_r[...] = jnp.maximum(hT, 0.0)


def _tc1(xT, agg, cnt16, W1l, b1l, W1r):
    return pl.pallas_call(
        _tc1_body,
        out_shape=jax.ShapeDtypeStruct((D_FEAT, NP), jnp.float32),
    )(xT, agg, cnt16, W1l, b1l.reshape(-1, 1), W1r)


def _tc2_body(h1T_r, agg_r, cnt_r,
              W2l_r, b2l_r, W2r_r, Wlin_r, blin_r, out_r):
    cnt = jnp.sum(cnt_r[...], axis=0, keepdims=True)
    inv = 1.0 / jnp.maximum(cnt, 1.0)
    meanT = agg_r[...] * inv
    h2T = _ddg(W2l_r[...], meanT) + b2l_r[...] + _ddg(W2r_r[...], h1T_r[...])
    h2T = jnp.maximum(h2T, 0.0)
    out_r[...] = _ddg(h2T, Wlin_r[...]) + blin_r[...]


def _tc2(h1T, agg, cnt16, W2l, b2l, W2r, Wlin, blin):
    n_cls = Wlin.shape[1]
    return pl.pallas_call(
        _tc2_body,
        out_shape=jax.ShapeDtypeStruct((NP, n_cls), jnp.float32),
    )(h1T, agg, cnt16, W2l, b2l.reshape(-1, 1), W2r,
      Wlin, blin.reshape(1, -1))


def kernel(x, edge_index, W1l, b1l, W1r, W2l, b2l, W2r, Wlin, blin):
    src = edge_index[0].astype(jnp.int32)
    dst = edge_index[1].astype(jnp.int32)
    pad = E_PAD - N_EDGES
    srcflat = jnp.concatenate([src, jnp.zeros((pad,), jnp.int32)])
    dstflat = jnp.concatenate([dst, jnp.full((pad,), N_NODES, jnp.int32)])

    xT = jnp.concatenate(
        [x.T, jnp.zeros((D_FEAT, NP - N_NODES), jnp.float32)], axis=1)
    xflat = xT.reshape(-1)

    aggf, cntf = _sc_agg1(xflat, srcflat, dstflat)
    agg = aggf.reshape(D_FEAT, NP)
    cnt16 = cntf.reshape(N_SUBCORES, NP)

    h1T = _tc1(xT, agg, cnt16, W1l, b1l, W1r)

    agg2 = _sc_agg2(h1T.reshape(-1), srcflat, dstflat).reshape(D_FEAT, NP)

    out = _tc2(h1T, agg2, cnt16, W2l, b2l, W2r, Wlin, blin)
    return out[:N_NODES]
